# baseline (speedup 1.0000x reference)
import functools

import jax
import jax.numpy as jnp
from jax import lax
from jax.experimental import pallas as pl
from jax.experimental.pallas import tpu as pltpu
from jax.experimental.pallas import tpu_sc as plsc

VOCAB = 30522
EMBED_DIM = 128
SEQ_LEN = 512
BATCH = 1024
FC_OUT = 64

QUARTER = 7680
PVIEW = 4 * QUARTER

NUM_CORES = 2
NUM_SUBCORES = 16
NW = NUM_CORES * NUM_SUBCORES
BPW = BATCH // NW
CHUNK = 128
CPB = SEQ_LEN // CHUNK
NCH = BPW * CPB
LANES = 16
NVR = FC_OUT // LANES

ROW_BLK = 2560


def _bf16_bits(u):
    return ((u + 0x7FFF + ((u >> 16) & 1)) >> 16) & 0xFFFF


def _project_kernel(q0_ref, q1_ref, q2_ref, q3_ref, w_ref, p2_ref):
    for q, ref in enumerate((q0_ref, q1_ref, q2_ref, q3_ref)):
        d = jnp.dot(ref[...], w_ref[...],
                    preferred_element_type=jnp.float32)
        u = jax.lax.bitcast_convert_type(d, jnp.int32)
        lo = _bf16_bits(u[:, 0:FC_OUT // 2])
        hi = _bf16_bits(u[:, FC_OUT // 2:FC_OUT])
        p2_ref[:, (FC_OUT // 2) * q:(FC_OUT // 2) * (q + 1)] = (
            lo | (hi << 16))


def _project(table, w):
    grid = (QUARTER // ROW_BLK,)
    qspec = lambda q: pl.BlockSpec(
        (ROW_BLK, EMBED_DIM),
        lambda i, q=q: (i + q * (QUARTER // ROW_BLK), 0))
    return pl.pallas_call(
        _project_kernel,
        grid=grid,
        in_specs=[qspec(0), qspec(1), qspec(2), qspec(3),
                  pl.BlockSpec((EMBED_DIM, FC_OUT), lambda i: (0, 0))],
        out_specs=pl.BlockSpec((ROW_BLK, 2 * FC_OUT), lambda i: (i, 0)),
        out_shape=jax.ShapeDtypeStruct((QUARTER, 2 * FC_OUT), jnp.int32),
    )(table, table, table, table, w)


_MESH = plsc.VectorSubcoreMesh(core_axis_name="c", subcore_axis_name="s",
                               num_cores=NUM_CORES, num_subcores=NUM_SUBCORES)


@functools.partial(
    pl.kernel,
    out_type=jax.ShapeDtypeStruct((BATCH, FC_OUT), jnp.float32),
    mesh=_MESH,
    scratch_types=[
        pltpu.VMEM((NCH, CHUNK), jnp.int32),
        pltpu.VMEM((2 * CPB, CHUNK, FC_OUT), jnp.bfloat16),
        pltpu.VMEM((BPW, FC_OUT), jnp.float32),
        pltpu.VMEM((FC_OUT,), jnp.float32),
        pltpu.SemaphoreType.DMA,
        pltpu.SemaphoreType.DMA,
        pltpu.SemaphoreType.DMA,
        pltpu.SemaphoreType.DMA,
        pltpu.SemaphoreType.DMA,
        pltpu.SemaphoreType.DMA,
        pltpu.SemaphoreType.DMA,
        pltpu.SemaphoreType.DMA,
    ],
    compiler_params=pltpu.CompilerParams(use_tc_tiling_on_sc=False,
                                         needs_layout_passes=False),
)
def _gather_mean(ids_hbm, p_hbm, b_hbm, out_hbm,
                 idx_v, rows_v, out_v, bias_v,
                 s0, s1, s2, s3, s4, s5, s6, s7):
    sems = (s0, s1, s2, s3, s4, s5, s6, s7)
    nbuf = 2 * CPB
    wid = lax.axis_index("s") * NUM_CORES + lax.axis_index("c")

    pltpu.sync_copy(b_hbm, bias_v)
    pltpu.sync_copy(ids_hbm.at[pl.ds(wid * NCH, NCH)], idx_v)

    def _copy(j, b):
        return pltpu.make_async_copy(
            p_hbm.at[idx_v.at[j]], rows_v.at[b], sems[b])

    for b in range(nbuf):
        _copy(b, b).start()

    inv = jnp.float32(1.0 / SEQ_LEN)
    off = [[16 * h, 32 + 16 * h] for h in range(2)]
    biases = [[bias_v[pl.ds(off[h][g], LANES)] for g in range(2)]
              for h in range(2)]

    def pair_body(bi2, _):
        for half in range(2):
            bi = bi2 * 2 + half
            accs = [jnp.zeros((LANES,), jnp.float32) for _ in range(NVR)]
            for c in range(CPB):
                b = half * CPB + c
                j = bi * CPB + c
                _copy(j, b).wait()

                def row_body(r4, a):
                    out = list(a)
                    r = r4 * 4
                    for h in range(2):
                        s = rows_v[b, r, pl.ds(32 * h, 32)]
                        for d in range(1, 4):
                            s = s + rows_v[b, r + d, pl.ds(32 * h, 32)]
                        e0, e1 = plsc.unpack(
                            s, format=plsc.PackFormat.INTERLEAVED)
                        out[2 * h] = out[2 * h] + e0
                        out[2 * h + 1] = out[2 * h + 1] + e1
                    return tuple(out)

                accs = list(lax.fori_loop(0, CHUNK // 4, row_body,
                                          tuple(accs), unroll=4))

                @pl.when(bi2 + 1 < BPW // 2)
                def _():
                    _copy(j + nbuf, b).start()

            for h in range(2):
                for g in range(2):
                    v = accs[2 * h + g] * inv + biases[h][g]
                    out_v[bi, pl.ds(off[h][g], LANES)] = jnp.maximum(v, 0.0)
        return 0

    lax.fori_loop(0, BPW // 2, pair_body, 0)
    pltpu.sync_copy(out_v, out_hbm.at[pl.ds(wid * BPW, BPW)])


def kernel(input_ids, attention_mask, emb_table, W, b):
    del attention_mask
    p2 = _project(emb_table, W)
    pview = jax.lax.bitcast_convert_type(p2, jnp.bfloat16).reshape(
        PVIEW, FC_OUT)
    ids = input_ids.astype(jnp.int32)
    idsv = ids * 4 - (4 * QUARTER - 1) * (ids // QUARTER)
    return _gather_mean(idsv.reshape(-1, CHUNK), pview, b)


# speedup vs baseline: 9.2796x; 9.2796x over previous
import functools

import jax
import jax.numpy as jnp
from jax import lax
from jax.experimental import pallas as pl
from jax.experimental.pallas import tpu as pltpu
from jax.experimental.pallas import tpu_sc as plsc

VOCAB = 30522
EMBED_DIM = 128
SEQ_LEN = 512
BATCH = 1024
FC_OUT = 64

QUARTER = 7680
PVIEW = 4 * QUARTER

NUM_CORES = 2
NUM_SUBCORES = 16
NW = NUM_CORES * NUM_SUBCORES
BPW = BATCH // NW
CHUNK = 128
CPB = SEQ_LEN // CHUNK
NCH = BPW * CPB
LANES = 16
NVR = FC_OUT // LANES

ROW_BLK = 2560


def _bf16_bits(u):
    return ((u + 0x7FFF + ((u >> 16) & 1)) >> 16) & 0xFFFF


def _project_kernel(q0_ref, q1_ref, q2_ref, q3_ref, w_ref, p2_ref):
    for q, ref in enumerate((q0_ref, q1_ref, q2_ref, q3_ref)):
        d = jnp.dot(ref[...], w_ref[...],
                    preferred_element_type=jnp.float32)
        u = jax.lax.bitcast_convert_type(d, jnp.int32)
        lo = _bf16_bits(u[:, 0:FC_OUT // 2])
        hi = _bf16_bits(u[:, FC_OUT // 2:FC_OUT])
        p2_ref[:, (FC_OUT // 2) * q:(FC_OUT // 2) * (q + 1)] = (
            lo | (hi << 16))


def _project(table, w):
    grid = (QUARTER // ROW_BLK,)
    qspec = lambda q: pl.BlockSpec(
        (ROW_BLK, EMBED_DIM),
        lambda i, q=q: (i + q * (QUARTER // ROW_BLK), 0))
    return pl.pallas_call(
        _project_kernel,
        grid=grid,
        in_specs=[qspec(0), qspec(1), qspec(2), qspec(3),
                  pl.BlockSpec((EMBED_DIM, FC_OUT), lambda i: (0, 0))],
        out_specs=pl.BlockSpec((ROW_BLK, 2 * FC_OUT), lambda i: (i, 0)),
        out_shape=jax.ShapeDtypeStruct((QUARTER, 2 * FC_OUT), jnp.int32),
    )(table, table, table, table, w)


_MESH = plsc.VectorSubcoreMesh(core_axis_name="c", subcore_axis_name="s",
                               num_cores=NUM_CORES, num_subcores=NUM_SUBCORES)


@functools.partial(
    pl.kernel,
    out_type=jax.ShapeDtypeStruct((BATCH, FC_OUT), jnp.float32),
    mesh=_MESH,
    scratch_types=[
        pltpu.VMEM((NCH, CHUNK), jnp.int32),
        pltpu.VMEM((2 * CPB, CHUNK, FC_OUT // 2), jnp.int32),
        pltpu.VMEM((BPW, FC_OUT), jnp.float32),
        pltpu.VMEM((FC_OUT,), jnp.float32),
        pltpu.SemaphoreType.DMA,
        pltpu.SemaphoreType.DMA,
        pltpu.SemaphoreType.DMA,
        pltpu.SemaphoreType.DMA,
        pltpu.SemaphoreType.DMA,
        pltpu.SemaphoreType.DMA,
        pltpu.SemaphoreType.DMA,
        pltpu.SemaphoreType.DMA,
    ],
    compiler_params=pltpu.CompilerParams(use_tc_tiling_on_sc=False,
                                         needs_layout_passes=False),
)
def _gather_mean(ids_hbm, p_hbm, b_hbm, out_hbm,
                 idx_v, rows_v, out_v, bias_v,
                 s0, s1, s2, s3, s4, s5, s6, s7):
    sems = (s0, s1, s2, s3, s4, s5, s6, s7)
    nbuf = 2 * CPB
    wid = lax.axis_index("s") * NUM_CORES + lax.axis_index("c")

    pltpu.sync_copy(b_hbm, bias_v)
    pltpu.sync_copy(ids_hbm.at[pl.ds(wid * NCH, NCH)], idx_v)

    def _copy(j, b):
        return pltpu.make_async_copy(
            p_hbm.at[idx_v.at[j]], rows_v.at[b], sems[b])

    for b in range(nbuf):
        _copy(b, b).start()

    inv = jnp.float32(1.0 / SEQ_LEN)
    off = [[16 * h, 32 + 16 * h] for h in range(2)]
    biases = [[bias_v[pl.ds(off[h][g], LANES)] for g in range(2)]
              for h in range(2)]

    def pair_body(bi2, _):
        for half in range(2):
            bi = bi2 * 2 + half
            accs = [jnp.zeros((LANES,), jnp.float32) for _ in range(NVR)]
            for c in range(CPB):
                b = half * CPB + c
                j = bi * CPB + c
                _copy(j, b).wait()

                def row_body(r4, a):
                    out = list(a)
                    r = r4 * 4
                    for h in range(2):
                        s = plsc.bitcast(
                            rows_v[b, r, pl.ds(LANES * h, LANES)],
                            jnp.bfloat16)
                        for d in range(1, 4):
                            s = s + plsc.bitcast(
                                rows_v[b, r + d, pl.ds(LANES * h, LANES)],
                                jnp.bfloat16)
                        e0, e1 = plsc.unpack(
                            s, format=plsc.PackFormat.INTERLEAVED)
                        out[2 * h] = out[2 * h] + e0
                        out[2 * h + 1] = out[2 * h + 1] + e1
                    return tuple(out)

                accs = list(lax.fori_loop(0, CHUNK // 4, row_body,
                                          tuple(accs), unroll=4))

                @pl.when(bi2 + 1 < BPW // 2)
                def _():
                    _copy(j + nbuf, b).start()

            for h in range(2):
                for g in range(2):
                    v = accs[2 * h + g] * inv + biases[h][g]
                    out_v[bi, pl.ds(off[h][g], LANES)] = jnp.maximum(v, 0.0)
        return 0

    lax.fori_loop(0, BPW // 2, pair_body, 0)
    pltpu.sync_copy(out_v, out_hbm.at[pl.ds(wid * BPW, BPW)])


def kernel(input_ids, attention_mask, emb_table, W, b):
    del attention_mask
    p2 = _project(emb_table, W)
    pview = p2.reshape(PVIEW, FC_OUT // 2)
    ids = input_ids.astype(jnp.int32)
    idsv = ids * 4 - (4 * QUARTER - 1) * (ids // QUARTER)
    return _gather_mean(idsv.reshape(-1, CHUNK), pview, b)
